# Initial kernel scaffold; baseline (speedup 1.0000x reference)
import jax
import jax.numpy as jnp
from jax.experimental import pallas as pl


def kernel(user_tensor, item_tensor, rand_index, item_content, W1, b1, W2, b2, user_emb, item_emb):
    raise NotImplementedError("write your pallas kernel here")


# speedup vs baseline: 1.1834x; 1.1834x over previous
import jax
import jax.numpy as jnp
from jax import lax
from jax.experimental import pallas as pl
from jax.experimental.pallas import tpu as pltpu
from jax.experimental.pallas import tpu_sc as plsc

ITEM_NUM = 100000
CONTENT_DIM = 128
HID = 256
EMB = 64
K = 20
TEMP = 2.0
LAM = 0.5
B = 4096
N_TOT = B * K
NC, NS, L = 2, 16, 16
NW = NC * NS
P_W = N_TOT // NW
NCHUNK = 20
CW = P_W // NCHUNK
NG = CW // L
NRAND = N_TOT // 2
RB = 10240


def _mlp_body(c_ref, w1_ref, b1_ref, w2_ref, b2_ref, out_ref):
    h = jnp.dot(c_ref[...], w1_ref[...], preferred_element_type=jnp.float32)
    h = h + b1_ref[...]
    h = jnp.where(h >= 0, h, 0.01 * h)
    out_ref[...] = (
        jnp.dot(h, w2_ref[...], preferred_element_type=jnp.float32) + b2_ref[...]
    )


def _encode(content, W1, b1, W2, b2):
    R = 800
    return pl.pallas_call(
        _mlp_body,
        grid=(ITEM_NUM // R,),
        in_specs=[
            pl.BlockSpec((R, CONTENT_DIM), lambda i: (i, 0)),
            pl.BlockSpec((CONTENT_DIM, HID), lambda i: (0, 0)),
            pl.BlockSpec((1, HID), lambda i: (0, 0)),
            pl.BlockSpec((HID, EMB), lambda i: (0, 0)),
            pl.BlockSpec((1, EMB), lambda i: (0, 0)),
        ],
        out_specs=pl.BlockSpec((R, EMB), lambda i: (i, 0)),
        out_shape=jax.ShapeDtypeStruct((ITEM_NUM, EMB), jnp.float32),
    )(content, W1, b1.reshape(1, HID), W2, b2.reshape(1, EMB))


def _sc_body(feat, item_emb, user_emb, idxi, idxu, posi, bidx, rand,
             d1_o, qf_o, qi_o, qu_o, d2_o,
             idxi_v, idxu_v, bidx_v, posi_v, epos_v, rand_v, mask_v,
             f_v, e_v, u_v, st_d1, st_qf, st_qi, st_qu, st_d2):
    wid = lax.axis_index("c") * NS + lax.axis_index("s")
    lo = wid * P_W

    pltpu.sync_copy(idxi.at[wid], idxi_v)
    pltpu.sync_copy(idxu.at[wid], idxu_v)
    pltpu.sync_copy(bidx, bidx_v)
    pltpu.sync_copy(posi.at[wid], posi_v)
    pltpu.sync_copy(item_emb.at[posi_v], epos_v)

    zero16f = jnp.zeros((L,), jnp.float32)
    ones16 = jnp.ones((L,), jnp.float32)
    iota16 = lax.iota(jnp.int32, L)

    @pl.loop(0, P_W // L)
    def _zero(i):
        mask_v[pl.ds(i * L, L)] = zero16f

    for rc in range(NRAND // RB):
        pltpu.sync_copy(rand.at[pl.ds(rc * RB, RB)], rand_v)

        @pl.loop(0, RB // L)
        def _scat(g):
            r16 = rand_v[pl.ds(g * L, L)]
            m = (r16 >= lo) & (r16 < lo + P_W)
            rl = jnp.where(m, r16 - lo, 0)
            plsc.store_scatter(mask_v, [rl], ones16, mask=m)

    @pl.loop(0, NCHUNK)
    def _chunk(j):
        pltpu.sync_copy(feat.at[idxi_v.at[j]], f_v)
        pltpu.sync_copy(item_emb.at[idxi_v.at[j]], e_v)
        pltpu.sync_copy(user_emb.at[idxu_v.at[j]], u_v)

        @pl.loop(0, NG)
        def _group(g):
            rows16 = iota16 + g * L
            bidx16 = bidx_v[j, pl.ds(g * L, L)]
            acc_d1 = zero16f
            acc_qf = zero16f
            acc_qi = zero16f
            acc_qu = zero16f
            acc_a = zero16f
            acc_b = zero16f
            for d in range(EMB):
                cd = jnp.full((L,), d, jnp.int32)
                fd = plsc.load_gather(f_v, [rows16, cd])
                ed = plsc.load_gather(e_v, [rows16, cd])
                ud = plsc.load_gather(u_v, [rows16, cd])
                pd = plsc.load_gather(epos_v, [bidx16, cd])
                acc_d1 = acc_d1 + pd * fd
                acc_qf = acc_qf + fd * fd
                acc_qi = acc_qi + ed * ed
                acc_qu = acc_qu + ud * ud
                acc_a = acc_a + ud * fd
                acc_b = acc_b + ud * ed
            m16 = mask_v[pl.ds(j * CW + g * L, L)]
            sl = pl.ds(g * L, L)
            st_d1[sl] = acc_d1
            st_qf[sl] = acc_qf
            st_qi[sl] = acc_qi
            st_qu[sl] = acc_qu
            st_d2[sl] = jnp.where(m16 > 0, acc_a, acc_b)

        pltpu.sync_copy(st_d1, d1_o.at[wid, j])
        pltpu.sync_copy(st_qf, qf_o.at[wid, j])
        pltpu.sync_copy(st_qi, qi_o.at[wid, j])
        pltpu.sync_copy(st_qu, qu_o.at[wid, j])
        pltpu.sync_copy(st_d2, d2_o.at[wid, j])


def _sc_call(feat, item_emb, user_emb, idxi, idxu, posi, bidx, rand):
    mesh = plsc.VectorSubcoreMesh(
        core_axis_name="c", subcore_axis_name="s", num_cores=NC, num_subcores=NS
    )
    out = jax.ShapeDtypeStruct((NW, NCHUNK, CW), jnp.float32)
    return pl.kernel(
        _sc_body,
        out_type=[out] * 5,
        mesh=mesh,
        compiler_params=pltpu.CompilerParams(
            needs_layout_passes=False, use_tc_tiling_on_sc=False
        ),
        scratch_types=[
            pltpu.VMEM((NCHUNK, CW), jnp.int32),
            pltpu.VMEM((NCHUNK, CW), jnp.int32),
            pltpu.VMEM((NCHUNK, CW), jnp.int32),
            pltpu.VMEM((CW,), jnp.int32),
            pltpu.VMEM((CW, EMB), jnp.float32),
            pltpu.VMEM((RB,), jnp.int32),
            pltpu.VMEM((P_W,), jnp.float32),
            pltpu.VMEM((CW, EMB), jnp.float32),
            pltpu.VMEM((CW, EMB), jnp.float32),
            pltpu.VMEM((CW, EMB), jnp.float32),
            pltpu.VMEM((CW,), jnp.float32),
            pltpu.VMEM((CW,), jnp.float32),
            pltpu.VMEM((CW,), jnp.float32),
            pltpu.VMEM((CW,), jnp.float32),
            pltpu.VMEM((CW,), jnp.float32),
        ],
    )(feat, item_emb, user_emb, idxi, idxu, posi, bidx, rand)


def _fin_body(d1, qf, qi, qu, d2, loss_o, reg_o):
    eps = 1e-12
    qi_all = qi[...]
    qpos = qi_all[:, 0:1]
    sc = jnp.maximum(jnp.sqrt(qpos), eps) * jnp.maximum(jnp.sqrt(qf[...]), eps)
    s1 = jnp.exp(d1[...] / (sc * TEMP))
    l1 = jnp.mean(
        jnp.log(jnp.sum(s1, axis=1, keepdims=True)) - jnp.log(s1[:, 0:1])
    )
    s2 = jnp.exp(d2[...] / TEMP)
    l2 = jnp.mean(
        jnp.log(jnp.sum(s2, axis=1, keepdims=True)) - jnp.log(s2[:, 0:1])
    )
    reg = (jnp.mean(jnp.sqrt(qu[...])) + jnp.mean(jnp.sqrt(qi_all))) / 2.0
    loss_o[...] = (l1 * LAM + l2 * (1.0 - LAM)).reshape(1, 1)
    reg_o[...] = reg.reshape(1, 1)


def _finish(d1, qf, qi, qu, d2):
    return pl.pallas_call(
        _fin_body,
        out_shape=[
            jax.ShapeDtypeStruct((1, 1), jnp.float32),
            jax.ShapeDtypeStruct((1, 1), jnp.float32),
        ],
    )(d1, qf, qi, qu, d2)


def kernel(user_tensor, item_tensor, rand_index, item_content, W1, b1, W2, b2,
           user_emb, item_emb):
    it = item_tensor.astype(jnp.int32)
    ut = user_tensor.astype(jnp.int32)
    feat = _encode(item_content, W1, b1, W2, b2)
    idxi = it.reshape(NW, NCHUNK, CW)
    idxu = ut.reshape(NW, NCHUNK, CW)
    posi = it[:, 0].reshape(NW, B // NW)
    bidx = (jnp.arange(P_W, dtype=jnp.int32) // K).reshape(NCHUNK, CW)
    rand = rand_index.astype(jnp.int32)
    d1, qf, qi, qu, d2 = _sc_call(
        feat, item_emb, user_emb, idxi, idxu, posi, bidx, rand
    )
    loss, reg = _finish(
        d1.reshape(B, K), qf.reshape(B, K), qi.reshape(B, K),
        qu.reshape(B, K), d2.reshape(B, K)
    )
    return loss.reshape(()), reg.reshape(())


# speedup vs baseline: 1.2847x; 1.0856x over previous
import jax
import jax.numpy as jnp
from jax import lax
from jax.experimental import pallas as pl
from jax.experimental.pallas import tpu as pltpu
from jax.experimental.pallas import tpu_sc as plsc

ITEM_NUM = 100000
CONTENT_DIM = 128
HID = 256
EMB = 64
K = 20
TEMP = 2.0
LAM = 0.5
B = 4096
N_TOT = B * K
NC, NS, L = 2, 16, 16
NW = NC * NS
P_W = N_TOT // NW
NCHUNK = 20
CW = P_W // NCHUNK
NG = CW // L
NRAND = N_TOT // 2
RB = 10240


def _mlp_body(c_ref, w1_ref, b1_ref, w2_ref, b2_ref, out_ref):
    h = jnp.dot(c_ref[...], w1_ref[...], preferred_element_type=jnp.float32)
    h = h + b1_ref[...]
    h = jnp.where(h >= 0, h, 0.01 * h)
    out_ref[...] = (
        jnp.dot(h, w2_ref[...], preferred_element_type=jnp.float32) + b2_ref[...]
    )


def _encode(content, W1, b1, W2, b2):
    R = 800
    return pl.pallas_call(
        _mlp_body,
        grid=(ITEM_NUM // R,),
        in_specs=[
            pl.BlockSpec((R, CONTENT_DIM), lambda i: (i, 0)),
            pl.BlockSpec((CONTENT_DIM, HID), lambda i: (0, 0)),
            pl.BlockSpec((1, HID), lambda i: (0, 0)),
            pl.BlockSpec((HID, EMB), lambda i: (0, 0)),
            pl.BlockSpec((1, EMB), lambda i: (0, 0)),
        ],
        out_specs=pl.BlockSpec((R, EMB), lambda i: (i, 0)),
        out_shape=jax.ShapeDtypeStruct((ITEM_NUM, EMB), jnp.float32),
    )(content, W1, b1.reshape(1, HID), W2, b2.reshape(1, EMB))


NBUF = 2


def _sc_body(feat, item_emb, user_emb, idxi, idxu, posi, bidx, rand,
             out_o,
             idxi_v, idxu_v, bidx_v, posi_v, epos_v, rand_v, mask_v,
             f0, f1, e0, e1, u0, u1, st,
             sf0, sf1, se0, se1, su0, su1, sp):
    wid = lax.axis_index("c") * NS + lax.axis_index("s")
    lo = wid * P_W
    fbuf = (f0, f1)
    ebuf = (e0, e1)
    ubuf = (u0, u1)
    fsem = (sf0, sf1)
    esem = (se0, se1)
    usem = (su0, su1)

    pltpu.sync_copy(idxi.at[wid], idxi_v)
    pltpu.sync_copy(idxu.at[wid], idxu_v)
    pltpu.sync_copy(bidx, bidx_v)
    pltpu.sync_copy(posi.at[wid], posi_v)
    ep_desc = pltpu.make_async_copy(item_emb.at[posi_v], epos_v, sp)
    ep_desc.start()

    def fire(j, db):
        pltpu.make_async_copy(feat.at[idxi_v.at[j]], fbuf[db], fsem[db]).start()
        pltpu.make_async_copy(item_emb.at[idxi_v.at[j]], ebuf[db], esem[db]).start()
        pltpu.make_async_copy(user_emb.at[idxu_v.at[j]], ubuf[db], usem[db]).start()

    def drain(j, db):
        pltpu.make_async_copy(feat.at[idxi_v.at[j]], fbuf[db], fsem[db]).wait()
        pltpu.make_async_copy(item_emb.at[idxi_v.at[j]], ebuf[db], esem[db]).wait()
        pltpu.make_async_copy(user_emb.at[idxu_v.at[j]], ubuf[db], usem[db]).wait()

    for db in range(NBUF):
        fire(db, db)

    zero16f = jnp.zeros((L,), jnp.float32)
    ones16 = jnp.ones((L,), jnp.float32)
    iota16 = lax.iota(jnp.int32, L)

    @pl.loop(0, P_W // L)
    def _zero(i):
        mask_v[pl.ds(i * L, L)] = zero16f

    for rc in range(NRAND // RB):
        pltpu.sync_copy(rand.at[pl.ds(rc * RB, RB)], rand_v)

        @pl.loop(0, RB // L)
        def _scat(g):
            r16 = rand_v[pl.ds(g * L, L)]
            m = (r16 >= lo) & (r16 < lo + P_W)
            rl = jnp.where(m, r16 - lo, 0)
            plsc.store_scatter(mask_v, [rl], ones16, mask=m)

    ep_desc.wait()

    @pl.loop(0, NCHUNK, step=NBUF)
    def _chunk(j0):
        for db in range(NBUF):
            j = j0 + db
            drain(j, db)
            f_v, e_v, u_v = fbuf[db], ebuf[db], ubuf[db]

            @pl.loop(0, NG)
            def _group(g):
                rows16 = iota16 + g * L
                bidx16 = bidx_v[j, pl.ds(g * L, L)]
                acc_d1 = zero16f
                acc_qf = zero16f
                acc_qi = zero16f
                acc_qu = zero16f
                acc_a = zero16f
                acc_b = zero16f
                for d in range(EMB):
                    cd = jnp.full((L,), d, jnp.int32)
                    fd = plsc.load_gather(f_v, [rows16, cd])
                    ed = plsc.load_gather(e_v, [rows16, cd])
                    ud = plsc.load_gather(u_v, [rows16, cd])
                    pd = plsc.load_gather(epos_v, [bidx16, cd])
                    acc_d1 = acc_d1 + pd * fd
                    acc_qf = acc_qf + fd * fd
                    acc_qi = acc_qi + ed * ed
                    acc_qu = acc_qu + ud * ud
                    acc_a = acc_a + ud * fd
                    acc_b = acc_b + ud * ed
                m16 = mask_v[pl.ds(j * CW + g * L, L)]
                sl = pl.ds(g * L, L)
                st[0, sl] = acc_d1
                st[1, sl] = acc_qf
                st[2, sl] = acc_qi
                st[3, sl] = acc_qu
                st[4, sl] = jnp.where(m16 > 0, acc_a, acc_b)

            @pl.when(j + NBUF < NCHUNK)
            def _():
                fire(j + NBUF, db)

            pltpu.sync_copy(st, out_o.at[wid, j])


def _sc_call(feat, item_emb, user_emb, idxi, idxu, posi, bidx, rand):
    mesh = plsc.VectorSubcoreMesh(
        core_axis_name="c", subcore_axis_name="s", num_cores=NC, num_subcores=NS
    )
    row = pltpu.VMEM((CW, EMB), jnp.float32)
    sem = pltpu.SemaphoreType.DMA
    return pl.kernel(
        _sc_body,
        out_type=jax.ShapeDtypeStruct((NW, NCHUNK, 5, CW), jnp.float32),
        mesh=mesh,
        compiler_params=pltpu.CompilerParams(
            needs_layout_passes=False, use_tc_tiling_on_sc=False
        ),
        scratch_types=[
            pltpu.VMEM((NCHUNK, CW), jnp.int32),
            pltpu.VMEM((NCHUNK, CW), jnp.int32),
            pltpu.VMEM((NCHUNK, CW), jnp.int32),
            pltpu.VMEM((CW,), jnp.int32),
            pltpu.VMEM((CW, EMB), jnp.float32),
            pltpu.VMEM((RB,), jnp.int32),
            pltpu.VMEM((P_W,), jnp.float32),
            row, row, row, row, row, row,
            pltpu.VMEM((5, CW), jnp.float32),
            sem, sem, sem, sem, sem, sem, sem,
        ],
    )(feat, item_emb, user_emb, idxi, idxu, posi, bidx, rand)


def _fin_body(d1, qf, qi, qu, d2, loss_o, reg_o):
    eps = 1e-12
    qi_all = qi[...]
    qpos = qi_all[:, 0:1]
    sc = jnp.maximum(jnp.sqrt(qpos), eps) * jnp.maximum(jnp.sqrt(qf[...]), eps)
    s1 = jnp.exp(d1[...] / (sc * TEMP))
    l1 = jnp.mean(
        jnp.log(jnp.sum(s1, axis=1, keepdims=True)) - jnp.log(s1[:, 0:1])
    )
    s2 = jnp.exp(d2[...] / TEMP)
    l2 = jnp.mean(
        jnp.log(jnp.sum(s2, axis=1, keepdims=True)) - jnp.log(s2[:, 0:1])
    )
    reg = (jnp.mean(jnp.sqrt(qu[...])) + jnp.mean(jnp.sqrt(qi_all))) / 2.0
    loss_o[...] = (l1 * LAM + l2 * (1.0 - LAM)).reshape(1, 1)
    reg_o[...] = reg.reshape(1, 1)


def _finish(d1, qf, qi, qu, d2):
    return pl.pallas_call(
        _fin_body,
        out_shape=[
            jax.ShapeDtypeStruct((1, 1), jnp.float32),
            jax.ShapeDtypeStruct((1, 1), jnp.float32),
        ],
    )(d1, qf, qi, qu, d2)


def kernel(user_tensor, item_tensor, rand_index, item_content, W1, b1, W2, b2,
           user_emb, item_emb):
    it = item_tensor.astype(jnp.int32)
    ut = user_tensor.astype(jnp.int32)
    feat = _encode(item_content, W1, b1, W2, b2)
    idxi = it.reshape(NW, NCHUNK, CW)
    idxu = ut.reshape(NW, NCHUNK, CW)
    posi = it[:, 0].reshape(NW, B // NW)
    bidx = (jnp.arange(P_W, dtype=jnp.int32) // K).reshape(NCHUNK, CW)
    rand = rand_index.astype(jnp.int32)
    o = _sc_call(feat, item_emb, user_emb, idxi, idxu, posi, bidx, rand)
    loss, reg = _finish(
        o[:, :, 0, :].reshape(B, K), o[:, :, 1, :].reshape(B, K),
        o[:, :, 2, :].reshape(B, K), o[:, :, 3, :].reshape(B, K),
        o[:, :, 4, :].reshape(B, K)
    )
    return loss.reshape(()), reg.reshape(())


# speedup vs baseline: 2.3535x; 1.8320x over previous
import jax
import jax.numpy as jnp
from jax import lax
from jax.experimental import pallas as pl
from jax.experimental.pallas import tpu as pltpu
from jax.experimental.pallas import tpu_sc as plsc

ITEM_NUM = 100000
CONTENT_DIM = 128
HID = 256
EMB = 64
K = 20
TEMP = 2.0
LAM = 0.5
B = 4096
N_TOT = B * K
NC, NS, L = 2, 16, 16
NW = NC * NS
P_W = N_TOT // NW
NCHUNK = 20
CW = P_W // NCHUNK
NG = CW // L
NRAND = N_TOT // 2
RB = 10240
NBUF = 2


def _mlp_body(c_ref, w1_ref, b1_ref, w2_ref, b2_ref, e_ref, out_ref):
    h = jnp.dot(c_ref[...], w1_ref[...], preferred_element_type=jnp.float32)
    h = h + b1_ref[...]
    h = jnp.where(h >= 0, h, 0.01 * h)
    f = jnp.dot(h, w2_ref[...], preferred_element_type=jnp.float32) + b2_ref[...]
    out_ref[...] = jnp.concatenate([f, e_ref[...]], axis=1)


def _encode(content, W1, b1, W2, b2, item_emb):
    R = 800
    return pl.pallas_call(
        _mlp_body,
        grid=(ITEM_NUM // R,),
        in_specs=[
            pl.BlockSpec((R, CONTENT_DIM), lambda i: (i, 0)),
            pl.BlockSpec((CONTENT_DIM, HID), lambda i: (0, 0)),
            pl.BlockSpec((1, HID), lambda i: (0, 0)),
            pl.BlockSpec((HID, EMB), lambda i: (0, 0)),
            pl.BlockSpec((1, EMB), lambda i: (0, 0)),
            pl.BlockSpec((R, EMB), lambda i: (i, 0)),
        ],
        out_specs=pl.BlockSpec((R, 2 * EMB), lambda i: (i, 0)),
        out_shape=jax.ShapeDtypeStruct((ITEM_NUM, 2 * EMB), jnp.float32),
    )(content, W1, b1.reshape(1, HID), W2, b2.reshape(1, EMB), item_emb)


def _sc_body(fe, u2, idxi, idxuh, idxuo, posi, bidx, rand,
             out_o,
             idxi_v, idxuh_v, idxuo_v, bidx_v, posi_v, ep_v, rand_v, mask_v,
             f0, f1, u0, u1, st,
             sf0, sf1, su0, su1, sp):
    wid = lax.axis_index("c") * NS + lax.axis_index("s")
    lo = wid * P_W
    febuf = (f0, f1)
    ubuf = (u0, u1)
    fsem = (sf0, sf1)
    usem = (su0, su1)

    pltpu.sync_copy(idxi.at[pl.ds(wid * NCHUNK, NCHUNK)], idxi_v)
    pltpu.sync_copy(idxuh.at[pl.ds(wid * NCHUNK, NCHUNK)], idxuh_v)
    pltpu.sync_copy(idxuo.at[pl.ds(wid * NCHUNK, NCHUNK)], idxuo_v)
    pltpu.sync_copy(bidx, bidx_v)
    pltpu.sync_copy(posi.at[wid], posi_v)
    ep_desc = pltpu.make_async_copy(fe.at[posi_v], ep_v, sp)
    ep_desc.start()

    def fire(j, db):
        pltpu.make_async_copy(fe.at[idxi_v.at[j]], febuf[db], fsem[db]).start()
        pltpu.make_async_copy(u2.at[idxuh_v.at[j]], ubuf[db], usem[db]).start()

    def drain(j, db):
        pltpu.make_async_copy(fe.at[idxi_v.at[j]], febuf[db], fsem[db]).wait()
        pltpu.make_async_copy(u2.at[idxuh_v.at[j]], ubuf[db], usem[db]).wait()

    for db in range(NBUF):
        fire(db, db)

    zero16f = jnp.zeros((L,), jnp.float32)
    ones16 = jnp.ones((L,), jnp.float32)
    iota16 = lax.iota(jnp.int32, L)

    @pl.loop(0, P_W // L)
    def _zero(i):
        mask_v[pl.ds(i * L, L)] = zero16f

    for rc in range(NRAND // RB):
        pltpu.sync_copy(rand.at[pl.ds(rc * RB, RB)], rand_v)

        @pl.loop(0, RB // L, unroll=4)
        def _scat(g):
            r16 = rand_v[pl.ds(g * L, L)]
            m = (r16 >= lo) & (r16 < lo + P_W)
            rl = jnp.where(m, r16 - lo, 0)
            plsc.store_scatter(mask_v, [rl], ones16, mask=m)

    ep_desc.wait()

    @pl.loop(0, NCHUNK, step=NBUF)
    def _chunk(j0):
        for db in range(NBUF):
            j = j0 + db
            drain(j, db)
            fe_v, u_v = febuf[db], ubuf[db]

            ones16i = jnp.full((L,), 1, jnp.int32)
            wrap16i = jnp.full((L,), EMB - 1, jnp.int32)
            c64 = jnp.full((L,), EMB, jnp.int32)

            @pl.loop(0, NG)
            def _group(g):
                rows16 = iota16 + g * L
                bidx16 = bidx_v[j, pl.ds(g * L, L)]
                uoff16 = idxuo_v[j, pl.ds(g * L, L)]
                m16 = mask_v[pl.ds(j * CW + g * L, L)] > 0.0
                acc_d1 = zero16f
                acc_qf = zero16f
                acc_qi = zero16f
                acc_qu = zero16f
                acc_d2 = zero16f
                cd = iota16
                for d in range(EMB):
                    cd64 = cd + c64
                    fd = plsc.load_gather(fe_v, [rows16, cd])
                    ed = plsc.load_gather(fe_v, [rows16, cd64])
                    ud = plsc.load_gather(u_v, [rows16, cd + uoff16])
                    pd = plsc.load_gather(ep_v, [bidx16, cd64])
                    wd = jnp.where(m16, fd, ed)
                    acc_d1 = acc_d1 + pd * fd
                    acc_qf = acc_qf + fd * fd
                    acc_qi = acc_qi + ed * ed
                    acc_qu = acc_qu + ud * ud
                    acc_d2 = acc_d2 + ud * wd
                    if d < EMB - 1:
                        cd = (cd + ones16i) & wrap16i
                sl = pl.ds(g * L, L)
                st[0, sl] = acc_d1
                st[1, sl] = acc_qf
                st[2, sl] = acc_qi
                st[3, sl] = acc_qu
                st[4, sl] = acc_d2

            @pl.when(j + NBUF < NCHUNK)
            def _():
                fire(j + NBUF, db)

            pltpu.sync_copy(st, out_o.at[pl.ds((wid * NCHUNK + j) * 5, 5)])


def _sc_call(fe, u2, idxi, idxuh, idxuo, posi, bidx, rand):
    mesh = plsc.VectorSubcoreMesh(
        core_axis_name="c", subcore_axis_name="s", num_cores=NC, num_subcores=NS
    )
    row = pltpu.VMEM((CW, 2 * EMB), jnp.float32)
    sem = pltpu.SemaphoreType.DMA
    return pl.kernel(
        _sc_body,
        out_type=jax.ShapeDtypeStruct((NW * NCHUNK * 5, CW), jnp.float32),
        mesh=mesh,
        compiler_params=pltpu.CompilerParams(
            needs_layout_passes=False, use_tc_tiling_on_sc=False
        ),
        scratch_types=[
            pltpu.VMEM((NCHUNK, CW), jnp.int32),
            pltpu.VMEM((NCHUNK, CW), jnp.int32),
            pltpu.VMEM((NCHUNK, CW), jnp.int32),
            pltpu.VMEM((NCHUNK, CW), jnp.int32),
            pltpu.VMEM((CW,), jnp.int32),
            pltpu.VMEM((CW, 2 * EMB), jnp.float32),
            pltpu.VMEM((RB,), jnp.int32),
            pltpu.VMEM((P_W,), jnp.float32),
            row, row, row, row,
            pltpu.VMEM((5, CW), jnp.float32),
            sem, sem, sem, sem, sem,
        ],
    )(fe, u2, idxi, idxuh, idxuo, posi, bidx, rand)


def _fin_body(d1, qf, qi, qu, d2, loss_o, reg_o):
    eps = 1e-12
    qi_all = qi[...]
    qpos = qi_all[:, 0:1]
    sc = jnp.maximum(jnp.sqrt(qpos), eps) * jnp.maximum(jnp.sqrt(qf[...]), eps)
    s1 = jnp.exp(d1[...] / (sc * TEMP))
    l1 = jnp.mean(
        jnp.log(jnp.sum(s1, axis=1, keepdims=True)) - jnp.log(s1[:, 0:1])
    )
    s2 = jnp.exp(d2[...] / TEMP)
    l2 = jnp.mean(
        jnp.log(jnp.sum(s2, axis=1, keepdims=True)) - jnp.log(s2[:, 0:1])
    )
    reg = (jnp.mean(jnp.sqrt(qu[...])) + jnp.mean(jnp.sqrt(qi_all))) / 2.0
    loss_o[...] = (l1 * LAM + l2 * (1.0 - LAM)).reshape(1, 1)
    reg_o[...] = reg.reshape(1, 1)


def _finish(d1, qf, qi, qu, d2):
    return pl.pallas_call(
        _fin_body,
        out_shape=[
            jax.ShapeDtypeStruct((1, 1), jnp.float32),
            jax.ShapeDtypeStruct((1, 1), jnp.float32),
        ],
    )(d1, qf, qi, qu, d2)


def kernel(user_tensor, item_tensor, rand_index, item_content, W1, b1, W2, b2,
           user_emb, item_emb):
    it = item_tensor.astype(jnp.int32)
    ut = user_tensor.astype(jnp.int32)
    fe = _encode(item_content, W1, b1, W2, b2, item_emb)
    u2 = user_emb.reshape(ITEM_NUM // 2, 2 * EMB)
    idxi = it.reshape(NW * NCHUNK, CW)
    uflat = ut.reshape(NW * NCHUNK, CW)
    idxuh = uflat // 2
    idxuo = (uflat % 2) * EMB
    posi = it[:, 0].reshape(NW, B // NW)
    bidx = (jnp.arange(P_W, dtype=jnp.int32) // K).reshape(NCHUNK, CW)
    rand = rand_index.astype(jnp.int32)
    o = _sc_call(fe, u2, idxi, idxuh, idxuo, posi, bidx, rand)
    o = o.reshape(NW * NCHUNK, 5, CW)
    loss, reg = _finish(
        o[:, 0, :].reshape(B, K), o[:, 1, :].reshape(B, K),
        o[:, 2, :].reshape(B, K), o[:, 3, :].reshape(B, K),
        o[:, 4, :].reshape(B, K)
    )
    return loss.reshape(()), reg.reshape(())


# speedup vs baseline: 2.3837x; 1.0128x over previous
import jax
import jax.numpy as jnp
from jax import lax
from jax.experimental import pallas as pl
from jax.experimental.pallas import tpu as pltpu
from jax.experimental.pallas import tpu_sc as plsc

ITEM_NUM = 100000
CONTENT_DIM = 128
HID = 256
EMB = 64
K = 20
TEMP = 2.0
LAM = 0.5
B = 4096
N_TOT = B * K
NC, NS, L = 2, 16, 16
NW = NC * NS
P_W = N_TOT // NW
NCHUNK = 20
CW = P_W // NCHUNK
NG = CW // L
NRAND = N_TOT // 2
RB = 10240
NBUF = 2


def _mlp_body(c_ref, w1_ref, b1_ref, w2_ref, b2_ref, e_ref, out_ref):
    h = jnp.dot(
        c_ref[...].astype(jnp.bfloat16),
        w1_ref[...].astype(jnp.bfloat16),
        preferred_element_type=jnp.float32,
    )
    h = h + b1_ref[...]
    h = jnp.where(h >= 0, h, 0.01 * h)
    f = jnp.dot(h, w2_ref[...], preferred_element_type=jnp.float32) + b2_ref[...]
    out_ref[...] = jnp.concatenate([f, e_ref[...]], axis=1)


def _encode(content, W1, b1, W2, b2, item_emb):
    R = 800
    return pl.pallas_call(
        _mlp_body,
        grid=(ITEM_NUM // R,),
        in_specs=[
            pl.BlockSpec((R, CONTENT_DIM), lambda i: (i, 0)),
            pl.BlockSpec((CONTENT_DIM, HID), lambda i: (0, 0)),
            pl.BlockSpec((1, HID), lambda i: (0, 0)),
            pl.BlockSpec((HID, EMB), lambda i: (0, 0)),
            pl.BlockSpec((1, EMB), lambda i: (0, 0)),
            pl.BlockSpec((R, EMB), lambda i: (i, 0)),
        ],
        out_specs=pl.BlockSpec((R, 2 * EMB), lambda i: (i, 0)),
        out_shape=jax.ShapeDtypeStruct((ITEM_NUM, 2 * EMB), jnp.float32),
    )(content, W1, b1.reshape(1, HID), W2, b2.reshape(1, EMB), item_emb)


def _sc_body(fe, u2, idxi, idxuh, idxuo, posi, bidx, rand,
             out_o,
             idxi_v, idxuh_v, idxuo_v, bidx_v, posi_v, ep_v, rand_v, mask_v,
             f0, f1, u0, u1, st,
             sf0, sf1, su0, su1, sp):
    wid = lax.axis_index("c") * NS + lax.axis_index("s")
    lo = wid * P_W
    febuf = (f0, f1)
    ubuf = (u0, u1)
    fsem = (sf0, sf1)
    usem = (su0, su1)

    pltpu.sync_copy(idxi.at[wid], idxi_v)
    pltpu.sync_copy(idxuh.at[wid], idxuh_v)
    pltpu.sync_copy(idxuo.at[wid], idxuo_v)
    pltpu.sync_copy(bidx, bidx_v)
    pltpu.sync_copy(posi.at[pl.ds(wid * CW, CW)], posi_v)
    ep_desc = pltpu.make_async_copy(fe.at[posi_v], ep_v, sp)
    ep_desc.start()

    def fire(j, db):
        idx_j = idxi_v.at[pl.ds(j * CW, CW)]
        uidx_j = idxuh_v.at[pl.ds(j * CW, CW)]
        pltpu.make_async_copy(fe.at[idx_j], febuf[db], fsem[db]).start()
        pltpu.make_async_copy(u2.at[uidx_j], ubuf[db], usem[db]).start()

    def drain(j, db):
        idx_j = idxi_v.at[pl.ds(j * CW, CW)]
        uidx_j = idxuh_v.at[pl.ds(j * CW, CW)]
        pltpu.make_async_copy(fe.at[idx_j], febuf[db], fsem[db]).wait()
        pltpu.make_async_copy(u2.at[uidx_j], ubuf[db], usem[db]).wait()

    for db in range(NBUF):
        fire(db, db)

    zero16f = jnp.zeros((L,), jnp.float32)
    ones16 = jnp.ones((L,), jnp.float32)
    iota16 = lax.iota(jnp.int32, L)

    @pl.loop(0, P_W // L)
    def _zero(i):
        mask_v[pl.ds(i * L, L)] = zero16f

    for rc in range(NRAND // RB):
        pltpu.sync_copy(rand.at[pl.ds(rc * RB, RB)], rand_v)

        @pl.loop(0, RB // L, unroll=4)
        def _scat(g):
            r16 = rand_v[pl.ds(g * L, L)]
            m = (r16 >= lo) & (r16 < lo + P_W)
            rl = jnp.where(m, r16 - lo, 0)
            plsc.store_scatter(mask_v, [rl], ones16, mask=m)

    ep_desc.wait()

    @pl.loop(0, NCHUNK, step=NBUF)
    def _chunk(j0):
        for db in range(NBUF):
            j = j0 + db
            drain(j, db)
            fe_v, u_v = febuf[db], ubuf[db]

            ones16i = jnp.full((L,), 1, jnp.int32)
            wrap16i = jnp.full((L,), EMB - 1, jnp.int32)
            c64 = jnp.full((L,), EMB, jnp.int32)

            @pl.loop(0, NG)
            def _group(g):
                rows16 = iota16 + g * L
                bidx16 = bidx_v[pl.ds(j * CW + g * L, L)]
                uoff16 = idxuo_v[pl.ds(j * CW + g * L, L)]
                m16 = mask_v[pl.ds(j * CW + g * L, L)] > 0.0
                acc_d1 = zero16f
                acc_qf = zero16f
                acc_qi = zero16f
                acc_qu = zero16f
                acc_d2 = zero16f
                cd = iota16
                for d in range(EMB):
                    cd64 = cd + c64
                    fd = plsc.load_gather(fe_v, [rows16, cd])
                    ed = plsc.load_gather(fe_v, [rows16, cd64])
                    ud = plsc.load_gather(u_v, [rows16, cd + uoff16])
                    pd = plsc.load_gather(ep_v, [bidx16, cd64])
                    wd = jnp.where(m16, fd, ed)
                    acc_d1 = acc_d1 + pd * fd
                    acc_qf = acc_qf + fd * fd
                    acc_qi = acc_qi + ed * ed
                    acc_qu = acc_qu + ud * ud
                    acc_d2 = acc_d2 + ud * wd
                    if d < EMB - 1:
                        cd = (cd + ones16i) & wrap16i
                for q in range(5):
                    sl = pl.ds(q * CW + g * L, L)
                    acc = (acc_d1, acc_qf, acc_qi, acc_qu, acc_d2)[q]
                    st[sl] = acc

            @pl.when(j + NBUF < NCHUNK)
            def _():
                fire(j + NBUF, db)

            pltpu.sync_copy(st, out_o.at[wid * NCHUNK + j])


def _sc_call(fe, u2, idxi, idxuh, idxuo, posi, bidx, rand):
    mesh = plsc.VectorSubcoreMesh(
        core_axis_name="c", subcore_axis_name="s", num_cores=NC, num_subcores=NS
    )
    row = pltpu.VMEM((CW, 2 * EMB), jnp.float32)
    sem = pltpu.SemaphoreType.DMA
    return pl.kernel(
        _sc_body,
        out_type=jax.ShapeDtypeStruct((NW * NCHUNK, 5 * CW), jnp.float32),
        mesh=mesh,
        compiler_params=pltpu.CompilerParams(
            needs_layout_passes=False, use_tc_tiling_on_sc=True
        ),
        scratch_types=[
            pltpu.VMEM((P_W,), jnp.int32),
            pltpu.VMEM((P_W,), jnp.int32),
            pltpu.VMEM((P_W,), jnp.int32),
            pltpu.VMEM((P_W,), jnp.int32),
            pltpu.VMEM((CW,), jnp.int32),
            pltpu.VMEM((CW, 2 * EMB), jnp.float32),
            pltpu.VMEM((RB,), jnp.int32),
            pltpu.VMEM((P_W,), jnp.float32),
            row, row, row, row,
            pltpu.VMEM((5 * CW,), jnp.float32),
            sem, sem, sem, sem, sem,
        ],
    )(fe, u2, idxi, idxuh, idxuo, posi, bidx, rand)


def _fin_body(d1, qf, qi, qu, d2, loss_o, reg_o):
    eps = 1e-12
    qi_all = qi[...]
    qpos = qi_all[:, 0:1]
    sc = jnp.maximum(jnp.sqrt(qpos), eps) * jnp.maximum(jnp.sqrt(qf[...]), eps)
    s1 = jnp.exp(d1[...] / (sc * TEMP))
    l1 = jnp.mean(
        jnp.log(jnp.sum(s1, axis=1, keepdims=True)) - jnp.log(s1[:, 0:1])
    )
    s2 = jnp.exp(d2[...] / TEMP)
    l2 = jnp.mean(
        jnp.log(jnp.sum(s2, axis=1, keepdims=True)) - jnp.log(s2[:, 0:1])
    )
    reg = (jnp.mean(jnp.sqrt(qu[...])) + jnp.mean(jnp.sqrt(qi_all))) / 2.0
    loss_o[...] = (l1 * LAM + l2 * (1.0 - LAM)).reshape(1, 1)
    reg_o[...] = reg.reshape(1, 1)


def _finish(d1, qf, qi, qu, d2):
    return pl.pallas_call(
        _fin_body,
        out_shape=[
            jax.ShapeDtypeStruct((1, 1), jnp.float32),
            jax.ShapeDtypeStruct((1, 1), jnp.float32),
        ],
    )(d1, qf, qi, qu, d2)


def kernel(user_tensor, item_tensor, rand_index, item_content, W1, b1, W2, b2,
           user_emb, item_emb):
    it = item_tensor.astype(jnp.int32)
    ut = user_tensor.astype(jnp.int32)
    fe = _encode(item_content, W1, b1, W2, b2, item_emb)
    u2 = user_emb.reshape(ITEM_NUM // 2, 2 * EMB)
    idxi = it.reshape(NW, P_W)
    uflat = ut.reshape(NW, P_W)
    idxuh = uflat // 2
    idxuo = (uflat % 2) * EMB
    posi = it[:, 0].astype(jnp.int32)
    bidx = jnp.arange(P_W, dtype=jnp.int32) // K
    rand = rand_index.astype(jnp.int32)
    o = _sc_call(fe, u2, idxi, idxuh, idxuo, posi, bidx, rand)
    o = o.reshape(NW * NCHUNK, 5, CW)
    loss, reg = _finish(
        o[:, 0, :].reshape(B, K), o[:, 1, :].reshape(B, K),
        o[:, 2, :].reshape(B, K), o[:, 3, :].reshape(B, K),
        o[:, 4, :].reshape(B, K)
    )
    return loss.reshape(()), reg.reshape(())
